# baseline (speedup 1.0000x reference)
import functools

import jax
import jax.numpy as jnp
from jax import lax
from jax.experimental import pallas as pl
from jax.experimental.pallas import tpu as pltpu
from jax.experimental.pallas import tpu_sc as plsc

VOCAB = 32128
EMB = 2048
HEAD = 64
MAXLEN = 2048

NC = 2
NS = 16
NW = NC * NS
LANES = 16

B = 4 * 2048
BPW = B // NW
CHUNK = 8
NCHUNK = BPW // CHUNK
NBUF = 4


def _sc_gather(tokens_flat, table, modality):
    mesh = plsc.VectorSubcoreMesh(core_axis_name="c", subcore_axis_name="s")

    SBUF = 3

    @functools.partial(
        pl.kernel,
        mesh=mesh,
        out_type=[
            jax.ShapeDtypeStruct((B, EMB), jnp.float32),
            jax.ShapeDtypeStruct((B,), jnp.int32),
        ],
        scratch_types=[
            pltpu.VMEM((BPW,), jnp.int32),
            *[pltpu.VMEM((CHUNK, EMB), jnp.float32) for _ in range(NBUF)],
            pltpu.VMEM((EMB,), jnp.float32),
            pltpu.VMEM((BPW,), jnp.int32),
            pltpu.VMEM_SHARED((NS, SBUF, CHUNK, EMB), jnp.float32),
            pltpu.SemaphoreType.DMA,
            *[pltpu.SemaphoreType.DMA for _ in range(NBUF)],
            *[pltpu.SemaphoreType.DMA for _ in range(SBUF)],
        ],
    )
    def k(tok_hbm, table_hbm, mod_hbm, x_hbm, mask_hbm, *scratch):
        idx_v = scratch[0]
        rows = scratch[1:1 + NBUF]
        mod_v = scratch[1 + NBUF]
        mask_v = scratch[2 + NBUF]
        spst = scratch[3 + NBUF]
        sem = scratch[4 + NBUF]
        ga = scratch[5 + NBUF:5 + 2 * NBUF]
        sp = scratch[5 + 2 * NBUF:5 + 2 * NBUF + SBUF]
        sid = lax.axis_index("s")
        wid = sid * NC + lax.axis_index("c")
        base = wid * BPW
        pltpu.sync_copy(tok_hbm.at[pl.ds(base, BPW)], idx_v)
        gacp = [None] * NBUF
        spcp = [None] * SBUF
        for c in range(NBUF):
            gacp[c] = pltpu.async_copy(
                table_hbm.at[idx_v.at[pl.ds(c * CHUNK, CHUNK)]],
                rows[c], ga[c])
        pltpu.sync_copy(mod_hbm, mod_v)
        one = jnp.full((LANES,), 1, jnp.int32)
        zero = jnp.full((LANES,), 0, jnp.int32)
        for u in range(BPW // LANES):
            t = idx_v[pl.ds(u * LANES, LANES)]
            mask_v[pl.ds(u * LANES, LANES)] = jnp.where(t > 0, one, zero)
        mkcp = pltpu.async_copy(mask_v, mask_hbm.at[pl.ds(base, BPW)], sem)

        for c in range(NCHUNK):
            b = c % NBUF
            sb = c % SBUF
            rv = rows[b]
            gacp[b].wait()
            nxt = c + 1
            if NBUF <= nxt < NCHUNK:
                nb = nxt % NBUF
                gacp[nb] = pltpu.async_copy(
                    table_hbm.at[idx_v.at[pl.ds(nxt * CHUNK, CHUNK)]],
                    rows[nb], ga[nb])
            def jbody(j, _):
                off = j * LANES
                mv = mod_v[pl.ds(off, LANES)]
                for r in range(CHUNK):
                    rv[r, pl.ds(off, LANES)] = rv[r, pl.ds(off, LANES)] + mv
                return 0

            lax.fori_loop(0, EMB // LANES, jbody, 0)
            if spcp[sb] is not None:
                spcp[sb].wait()
            pltpu.sync_copy(rv, spst.at[sid, sb])
            spcp[sb] = pltpu.async_copy(
                spst.at[sid, sb],
                x_hbm.at[pl.ds(base + c * CHUNK, CHUNK)], sp[sb])
        for s in range(SBUF):
            if spcp[s] is not None:
                spcp[s].wait()
        mkcp.wait()

    return k(tokens_flat, table, modality)


def _rope_cache():
    inv_freq = 1.0 / (10000.0 ** (
        jnp.arange(0, HEAD, 2, dtype=jnp.float32) / HEAD))
    t = jnp.arange(MAXLEN, dtype=jnp.float32)
    freqs = jnp.einsum('i,j->ij', t, inv_freq)
    return jnp.concatenate([jnp.cos(freqs), jnp.sin(freqs)], axis=-1)


def _pos_broadcast(cache64, bs):
    blk = 256
    reps = EMB // HEAD

    def body(c_ref, o_ref):
        c = c_ref[...]
        o_ref[...] = jnp.reshape(
            jnp.broadcast_to(c[None, :, None, :], (bs, blk, reps, HEAD)),
            (bs, blk, EMB))

    return pl.pallas_call(
        body,
        grid=(MAXLEN // blk,),
        in_specs=[pl.BlockSpec((blk, HEAD), lambda i: (i, 0))],
        out_specs=pl.BlockSpec((bs, blk, EMB), lambda i: (0, i, 0)),
        out_shape=jax.ShapeDtypeStruct((bs, MAXLEN, EMB), jnp.float32),
    )(cache64)


def kernel(tokens, shared_embed, modality_embedding):
    bs, seq = tokens.shape
    pos_emb = _pos_broadcast(_rope_cache(), bs)
    x_flat, mask_flat = _sc_gather(tokens.reshape(-1), shared_embed,
                                   modality_embedding)
    return (x_flat.reshape(bs, seq, EMB), mask_flat.reshape(bs, seq), pos_emb)


# speedup vs baseline: 1.2438x; 1.2438x over previous
import functools

import jax
import jax.numpy as jnp
from jax import lax
from jax.experimental import pallas as pl
from jax.experimental.pallas import tpu as pltpu
from jax.experimental.pallas import tpu_sc as plsc

VOCAB = 32128
EMB = 2048
HEAD = 64
MAXLEN = 2048

NC = 2
NS = 16
NW = NC * NS
LANES = 16

B = 4 * 2048
BPW = B // NW
CHUNK = 16
NCHUNK = BPW // CHUNK
NBUF = 3


def _sc_gather(tokens_flat, table, modality):
    mesh = plsc.VectorSubcoreMesh(core_axis_name="c", subcore_axis_name="s")

    @functools.partial(
        pl.kernel,
        mesh=mesh,
        out_type=[
            jax.ShapeDtypeStruct((B, EMB), jnp.float32),
            jax.ShapeDtypeStruct((B,), jnp.int32),
        ],
        scratch_types=[
            pltpu.VMEM((BPW,), jnp.int32),
            *[pltpu.VMEM((CHUNK, EMB), jnp.float32) for _ in range(NBUF)],
            pltpu.VMEM((EMB,), jnp.float32),
            pltpu.VMEM((BPW,), jnp.int32),
            pltpu.SemaphoreType.DMA,
            *[pltpu.SemaphoreType.DMA for _ in range(NBUF)],
            *[pltpu.SemaphoreType.DMA for _ in range(NBUF)],
        ],
    )
    def k(tok_hbm, table_hbm, mod_hbm, x_hbm, mask_hbm, *scratch):
        idx_v = scratch[0]
        rows = scratch[1:1 + NBUF]
        mod_v = scratch[1 + NBUF]
        mask_v = scratch[2 + NBUF]
        sem = scratch[3 + NBUF]
        ga = scratch[4 + NBUF:4 + 2 * NBUF]
        wo = scratch[4 + 2 * NBUF:4 + 3 * NBUF]
        wid = lax.axis_index("s") * NC + lax.axis_index("c")
        base = wid * BPW
        pltpu.sync_copy(tok_hbm.at[pl.ds(base, BPW)], idx_v)
        gacp = [None] * NBUF
        wocp = [None] * NBUF
        for c in range(NBUF):
            gacp[c] = pltpu.async_copy(
                table_hbm.at[idx_v.at[pl.ds(c * CHUNK, CHUNK)]],
                rows[c], ga[c])
        pltpu.sync_copy(mod_hbm, mod_v)
        one = jnp.full((LANES,), 1, jnp.int32)
        zero = jnp.full((LANES,), 0, jnp.int32)
        for u in range(BPW // LANES):
            t = idx_v[pl.ds(u * LANES, LANES)]
            mask_v[pl.ds(u * LANES, LANES)] = jnp.where(t > 0, one, zero)
        mkcp = pltpu.async_copy(mask_v, mask_hbm.at[pl.ds(base, BPW)], sem)

        for c in range(NCHUNK):
            b = c % NBUF
            rv = rows[b]
            gacp[b].wait()
            nxt = c + 1
            if NBUF <= nxt < NCHUNK:
                nb = nxt % NBUF
                wocp[nb].wait()
                gacp[nb] = pltpu.async_copy(
                    table_hbm.at[idx_v.at[pl.ds(nxt * CHUNK, CHUNK)]],
                    rows[nb], ga[nb])
            def jbody(j, _):
                off = j * LANES
                mv = mod_v[pl.ds(off, LANES)]
                for r in range(CHUNK):
                    rv[r, pl.ds(off, LANES)] = rv[r, pl.ds(off, LANES)] + mv
                return 0

            lax.fori_loop(0, EMB // LANES, jbody, 0)
            wocp[b] = pltpu.async_copy(
                rv, x_hbm.at[pl.ds(base + c * CHUNK, CHUNK)], wo[b])
        for c in range(NCHUNK - min(NBUF, NCHUNK), NCHUNK):
            wocp[c % NBUF].wait()
        mkcp.wait()

    return k(tokens_flat, table, modality)


def _rope_cache():
    inv_freq = 1.0 / (10000.0 ** (
        jnp.arange(0, HEAD, 2, dtype=jnp.float32) / HEAD))
    t = jnp.arange(MAXLEN, dtype=jnp.float32)
    freqs = jnp.einsum('i,j->ij', t, inv_freq)
    return jnp.concatenate([jnp.cos(freqs), jnp.sin(freqs)], axis=-1)


def _pos_broadcast(cache64, bs):
    blk = 256
    reps = EMB // HEAD

    def body(c_ref, o_ref):
        c = c_ref[...]
        o_ref[...] = jnp.reshape(
            jnp.broadcast_to(c[None, :, None, :], (bs, blk, reps, HEAD)),
            (bs, blk, EMB))

    return pl.pallas_call(
        body,
        grid=(MAXLEN // blk,),
        in_specs=[pl.BlockSpec((blk, HEAD), lambda i: (i, 0))],
        out_specs=pl.BlockSpec((bs, blk, EMB), lambda i: (0, i, 0)),
        out_shape=jax.ShapeDtypeStruct((bs, MAXLEN, EMB), jnp.float32),
    )(cache64)


def kernel(tokens, shared_embed, modality_embedding):
    bs, seq = tokens.shape
    pos_emb = _pos_broadcast(_rope_cache(), bs)
    x_flat, mask_flat = _sc_gather(tokens.reshape(-1), shared_embed,
                                   modality_embedding)
    return (x_flat.reshape(bs, seq, EMB), mask_flat.reshape(bs, seq), pos_emb)
